# baseline (speedup 1.0000x reference)
import functools

import jax
import jax.numpy as jnp
from jax import lax
from jax.experimental import pallas as pl
from jax.experimental.pallas import tpu as pltpu
from jax.experimental.pallas import tpu_sc as plsc

EMB = 16
N_WORD = 200
ROWS_PER_W = 8
N_WORD_WORKERS = N_WORD // ROWS_PER_W
OUT_ROWS = 8 + N_WORD
IN_DIM = 3217
HID = 128


def _sc_gather(word_hbm, spk_hbm, widx_hbm, sidx_hbm, out_hbm, idx_v, rows_v, sem):
    info = plsc.get_sparse_core_info()
    nc = info.num_cores
    wid = lax.axis_index("s") * nc + lax.axis_index("c")

    @pl.when(wid == 0)
    def _():
        pltpu.sync_copy(sidx_hbm, idx_v)
        pltpu.async_copy(spk_hbm.at[idx_v], rows_v, sem).wait()
        pltpu.sync_copy(rows_v, out_hbm.at[pl.ds(0, ROWS_PER_W)])

    @pl.when((wid >= 1) & (wid <= N_WORD_WORKERS))
    def _():
        base = pl.multiple_of((wid - 1) * ROWS_PER_W, ROWS_PER_W)
        pltpu.sync_copy(widx_hbm.at[pl.ds(base, ROWS_PER_W)], idx_v)
        pltpu.async_copy(word_hbm.at[idx_v], rows_v, sem).wait()
        obase = pl.multiple_of(wid * ROWS_PER_W, ROWS_PER_W)
        pltpu.sync_copy(rows_v, out_hbm.at[pl.ds(obase, ROWS_PER_W)])


def _mlp_kernel(x_ref, w1_ref, b1_ref, w2c_ref, b2_ref, o_ref):
    h = lax.dot_general(
        x_ref[...], w1_ref[...], (((1,), (1,)), ((), ())),
        preferred_element_type=jnp.float32)
    h = jnp.maximum(h + b1_ref[...], 0.0)
    o = lax.dot_general(
        h, w2c_ref[...], (((1,), (0,)), ((), ())),
        preferred_element_type=jnp.float32)
    o_ref[...] = jax.nn.sigmoid(o + b2_ref[...])


def kernel(context_indices, speaker, col_three_indices, quant, sentiment,
           word_emb, speaker_emb, W1, b1, W2, b2):
    del sentiment
    widx = jnp.concatenate(
        [context_indices.astype(jnp.int32), col_three_indices.astype(jnp.int32)])
    sidx = jnp.broadcast_to(speaker.astype(jnp.int32), (ROWS_PER_W,))

    mesh = plsc.VectorSubcoreMesh(core_axis_name="c", subcore_axis_name="s")
    gathered = pl.kernel(
        _sc_gather,
        mesh=mesh,
        compiler_params=pltpu.CompilerParams(use_tc_tiling_on_sc=False),
        out_type=jax.ShapeDtypeStruct((OUT_ROWS, EMB), jnp.float32),
        scratch_types=[
            pltpu.VMEM((ROWS_PER_W,), jnp.int32),
            pltpu.VMEM((ROWS_PER_W, EMB), jnp.float32),
            pltpu.SemaphoreType.DMA,
        ],
    )(word_emb, speaker_emb, widx, sidx)

    x = jnp.concatenate(
        [gathered[7:].reshape(1, (N_WORD + 1) * EMB),
         quant.reshape(1, 1).astype(jnp.float32)], axis=1)

    w2c = jnp.pad(W2.reshape(HID, 1), ((0, 0), (0, HID - 1)))
    b2v = jnp.broadcast_to(b2.reshape(1, 1), (1, HID))
    out = pl.pallas_call(
        _mlp_kernel,
        out_shape=jax.ShapeDtypeStruct((1, HID), jnp.float32),
    )(x, W1, b1.reshape(1, HID), w2c, b2v)
    return out[:, :1]
